# baseline (speedup 1.0000x reference)
import jax
import jax.numpy as jnp
from jax import lax
from jax.experimental import pallas as pl
from jax.experimental.pallas import tpu as pltpu
from jax.experimental.pallas import tpu_sc as plsc

_NC = 2
_NS = 16
_NW = _NC * _NS
_CHUNK = 128
_K = 2
_NBUF = 3


def _gather_body(nslabs, x_hbm, tab_hbm, out_hbm, idx_v, *rest):
    bufs = rest[:_NBUF]
    gsems = rest[_NBUF:2 * _NBUF]
    wsems = rest[2 * _NBUF:3 * _NBUF]
    wid = lax.axis_index("s") * _NC + lax.axis_index("c")
    base = wid * nslabs
    pltpu.sync_copy(x_hbm.at[wid], idx_v)

    def start_gather(s, b):
        for k in range(_K):
            pltpu.async_copy(tab_hbm.at[idx_v.at[s, k]], bufs[b].at[k],
                             gsems[b])

    def wait_gather(b):
        for k in range(_K):
            pltpu.make_async_copy(tab_hbm.at[idx_v.at[0, 0]], bufs[b].at[k],
                                  gsems[b]).wait()

    def start_write(s, b):
        pltpu.async_copy(bufs[b], out_hbm.at[pl.ds((base + s) * _K, _K)],
                         wsems[b])

    def wait_write(b):
        pltpu.make_async_copy(bufs[b], out_hbm.at[pl.ds(0, _K)],
                              wsems[b]).wait()

    for b in range(_NBUF):
        start_gather(b, b)

    nrounds = -(-nslabs // _NBUF)

    def outer(t, carry):
        for b in range(_NBUF):
            s = t * _NBUF + b

            @pl.when(s < nslabs)
            def _():
                wait_gather(b)
                start_write(s, b)

        for b in range(_NBUF):
            sn = (t + 1) * _NBUF + b

            @pl.when(sn < nslabs)
            def _():
                wait_write(b)
                start_gather(sn, b)

        return carry

    lax.fori_loop(0, nrounds, outer, 0)

    for b in range(_NBUF):
        wait_write(b)


def kernel(x, weight):
    B, T = x.shape
    V, D = weight.shape
    n = B * T
    assert n % (_NW * _CHUNK * _K) == 0
    nslabs = n // (_NW * _CHUNK * _K)

    x4 = x.reshape(_NW, nslabs, _K, _CHUNK).astype(jnp.int32)
    mesh = plsc.VectorSubcoreMesh(core_axis_name="c", subcore_axis_name="s")

    body = lambda *refs: _gather_body(nslabs, *refs)
    out = pl.kernel(
        body,
        out_type=jax.ShapeDtypeStruct((n // _CHUNK, _CHUNK, D), jnp.float32),
        mesh=mesh,
        scratch_types=(
            [pltpu.VMEM((nslabs, _K, _CHUNK), jnp.int32)]
            + [pltpu.VMEM((_K, _CHUNK, D), jnp.float32)
               for _ in range(_NBUF)]
            + [pltpu.SemaphoreType.DMA for _ in range(2 * _NBUF)]
        ),
    )(x4, weight)
    return out.reshape(B, T, D)


# speedup vs baseline: 1.7509x; 1.7509x over previous
import jax
import jax.numpy as jnp
from jax import lax
from jax.experimental import pallas as pl
from jax.experimental.pallas import tpu as pltpu
from jax.experimental.pallas import tpu_sc as plsc

_NC = 2
_NS = 16
_NW = _NC * _NS
_NB = 8
_NBUF = 2


def _gather_body(nb_per_w, T, x_hbm, tab_hbm, out_hbm, idx_v, *rest):
    bufs = rest[:_NBUF]
    gsems = rest[_NBUF:2 * _NBUF]
    wsems = rest[2 * _NBUF:3 * _NBUF]
    nslabs = nb_per_w // _NB
    wid = lax.axis_index("s") * _NC + lax.axis_index("c")
    bbase = pl.multiple_of(wid * nb_per_w, 8)
    pltpu.sync_copy(x_hbm.at[pl.ds(bbase, nb_per_w)], idx_v)

    def start_gather(s, b):
        for i in range(_NB):
            pltpu.async_copy(tab_hbm.at[idx_v.at[s * _NB + i]],
                             bufs[b].at[i], gsems[b])

    def wait_gather(b):
        for i in range(_NB):
            pltpu.make_async_copy(tab_hbm.at[idx_v.at[0]], bufs[b].at[i],
                                  gsems[b]).wait()

    def start_write(s, b):
        off = pl.multiple_of(bbase + s * _NB, 8)
        pltpu.async_copy(bufs[b], out_hbm.at[pl.ds(off, _NB)], wsems[b])

    def wait_write(b):
        pltpu.make_async_copy(bufs[b], out_hbm.at[pl.ds(0, _NB)],
                              wsems[b]).wait()

    for b in range(_NBUF):
        start_gather(b, b)

    def outer(t, carry):
        for b in range(_NBUF):
            s = t * _NBUF + b
            wait_gather(b)
            start_write(s, b)
        for b in range(_NBUF):
            sn = (t + 1) * _NBUF + b

            @pl.when(sn < nslabs)
            def _():
                wait_write(b)
                start_gather(sn, b)

        return carry

    lax.fori_loop(0, nslabs // _NBUF, outer, 0)

    for b in range(_NBUF):
        wait_write(b)


def kernel(x, weight):
    B, T = x.shape
    V, D = weight.shape
    assert B % (_NW * _NB) == 0
    nb_per_w = B // _NW

    xi = x.astype(jnp.int32)
    mesh = plsc.VectorSubcoreMesh(core_axis_name="c", subcore_axis_name="s")

    body = lambda *refs: _gather_body(nb_per_w, T, *refs)
    out = pl.kernel(
        body,
        out_type=jax.ShapeDtypeStruct((B, T, D), jnp.float32),
        mesh=mesh,
        scratch_types=(
            [pltpu.VMEM((nb_per_w, T), jnp.int32)]
            + [pltpu.VMEM((_NB, T, D), jnp.float32) for _ in range(_NBUF)]
            + [pltpu.SemaphoreType.DMA for _ in range(2 * _NBUF)]
        ),
    )(xi, weight)
    return out


# speedup vs baseline: 3.2060x; 1.8310x over previous
import jax
import jax.numpy as jnp
from jax import lax
from jax.experimental import pallas as pl
from jax.experimental.pallas import tpu as pltpu
from jax.experimental.pallas import tpu_sc as plsc

_NC = 2
_NS = 16
_NW = _NC * _NS
_CHUNK = 128
_NBUF = 5


def _gather_body(T, xT_hbm, tab_hbm, out_hbm, idx_v, *rest):
    bufs = rest[:_NBUF]
    gsems = rest[_NBUF:2 * _NBUF]
    wsems = rest[2 * _NBUF:3 * _NBUF]
    wid = lax.axis_index("s") * _NC + lax.axis_index("c")
    coff = pl.multiple_of(wid * _CHUNK, 8)
    pltpu.sync_copy(xT_hbm.at[:, pl.ds(coff, _CHUNK)], idx_v)

    def start_gather(t, b):
        pltpu.async_copy(tab_hbm.at[idx_v.at[t]], bufs[b], gsems[b])

    def wait_gather(b):
        pltpu.make_async_copy(tab_hbm.at[idx_v.at[0]], bufs[b],
                              gsems[b]).wait()

    def start_write(t, b):
        pltpu.async_copy(bufs[b], out_hbm.at[t, pl.ds(coff, _CHUNK)],
                         wsems[b])

    def wait_write(b):
        pltpu.make_async_copy(bufs[b], out_hbm.at[0, pl.ds(0, _CHUNK)],
                              wsems[b]).wait()

    for b in range(_NBUF):
        start_gather(b, b)

    def outer(r, carry):
        for b in range(_NBUF):
            t = r * _NBUF + b
            wait_gather(b)
            start_write(t, b)
        for b in range(_NBUF):
            tn = (r + 1) * _NBUF + b

            @pl.when(tn < T)
            def _():
                wait_write(b)
                start_gather(tn, b)

        return carry

    lax.fori_loop(0, T // _NBUF, outer, 0)

    for b in range(_NBUF):
        wait_write(b)


def kernel(x, weight):
    B, T = x.shape
    V, D = weight.shape
    assert B % (_NW * _CHUNK) == 0 and T % _NBUF == 0

    xT = x.T.astype(jnp.int32)
    mesh = plsc.VectorSubcoreMesh(core_axis_name="c", subcore_axis_name="s")

    body = lambda *refs: _gather_body(T, *refs)
    out = pl.kernel(
        body,
        out_type=jax.ShapeDtypeStruct((T, B, D), jnp.float32),
        mesh=mesh,
        scratch_types=(
            [pltpu.VMEM((T, _CHUNK), jnp.int32)]
            + [pltpu.VMEM((_CHUNK, D), jnp.float32) for _ in range(_NBUF)]
            + [pltpu.SemaphoreType.DMA for _ in range(2 * _NBUF)]
        ),
    )(xT, weight)
    return out.transpose(1, 0, 2)


# speedup vs baseline: 3.2216x; 1.0049x over previous
import jax
import jax.numpy as jnp
from jax import lax
from jax.experimental import pallas as pl
from jax.experimental.pallas import tpu as pltpu
from jax.experimental.pallas import tpu_sc as plsc

_NC = 2
_NS = 16
_NW = _NC * _NS
_CHUNK = 128
_NBUF = 7


def _gather_body(T, xT_hbm, tab_hbm, out_hbm, idx_v, *rest):
    bufs = rest[:_NBUF]
    gsems = rest[_NBUF:2 * _NBUF]
    wsems = rest[2 * _NBUF:3 * _NBUF]
    wid = lax.axis_index("s") * _NC + lax.axis_index("c")
    coff = pl.multiple_of(wid * _CHUNK, 8)
    pltpu.sync_copy(xT_hbm.at[:, pl.ds(coff, _CHUNK)], idx_v)

    def start_gather(t, b):
        pltpu.async_copy(tab_hbm.at[idx_v.at[t]], bufs[b], gsems[b])

    def wait_gather(b):
        pltpu.make_async_copy(tab_hbm.at[idx_v.at[0]], bufs[b],
                              gsems[b]).wait()

    def start_write(t, b):
        pltpu.async_copy(bufs[b], out_hbm.at[t, pl.ds(coff, _CHUNK)],
                         wsems[b])

    def wait_write(b):
        pltpu.make_async_copy(bufs[b], out_hbm.at[0, pl.ds(0, _CHUNK)],
                              wsems[b]).wait()

    for b in range(_NBUF):
        start_gather(b, b)

    def outer(r, carry):
        for b in range(_NBUF):
            t = r * _NBUF + b

            @pl.when(t < T)
            def _():
                wait_gather(b)
                start_write(t, b)

        for b in range(_NBUF):
            tn = (r + 1) * _NBUF + b

            @pl.when(tn < T)
            def _():
                wait_write(b)
                start_gather(tn, b)

        return carry

    lax.fori_loop(0, -(-T // _NBUF), outer, 0)

    for b in range(_NBUF):
        wait_write(b)


def kernel(x, weight):
    B, T = x.shape
    V, D = weight.shape
    assert B % (_NW * _CHUNK) == 0 and T >= _NBUF

    xT = x.T.astype(jnp.int32)
    mesh = plsc.VectorSubcoreMesh(core_axis_name="c", subcore_axis_name="s")

    body = lambda *refs: _gather_body(T, *refs)
    out = pl.kernel(
        body,
        out_type=jax.ShapeDtypeStruct((T, B, D), jnp.float32),
        mesh=mesh,
        scratch_types=(
            [pltpu.VMEM((T, _CHUNK), jnp.int32)]
            + [pltpu.VMEM((_CHUNK, D), jnp.float32) for _ in range(_NBUF)]
            + [pltpu.SemaphoreType.DMA for _ in range(2 * _NBUF)]
        ),
    )(xT, weight)
    return out.transpose(1, 0, 2)
